# baseline (device time: 21231 ns/iter reference)
import jax
import jax.numpy as jnp
from jax import lax
from jax.experimental import pallas as pl
from jax.experimental.pallas import tpu as pltpu


def kernel(ids, E):
    v_per, d = E.shape
    t = ids.shape[0]
    q = t // 4
    h = q // 2

    def body(ids_s, ids_v, e_ref, out_ref, gbuf, fresh_ref,
             gsem, y_ssem, y_rsem, f_ssem, f_rsem):
        mx = lax.axis_index("x")
        my = lax.axis_index("y")
        mz = lax.axis_index("z")
        y_nbr = (mx, 1 - my, mz)
        x_nbr = (1 - mx, my, mz)
        z_nbr = (mx, my, 1 - mz)
        aa = 2 * mx + mz
        bb = 3 - aa
        base = my * v_per

        barrier = pltpu.get_barrier_semaphore()
        for nbr in (y_nbr, x_nbr, z_nbr):
            pl.semaphore_signal(
                barrier, inc=1, device_id=nbr,
                device_id_type=pl.DeviceIdType.MESH,
            )

        def issue_rows(n, src_base, dst_base, sem):
            def one(i, _):
                lid = jnp.bitwise_and(ids_s[src_base + i] - base, v_per - 1)
                pltpu.make_async_copy(
                    e_ref.at[pl.ds(lid, 1), :],
                    gbuf.at[pl.ds(dst_base + i, 1), :],
                    sem,
                ).start()
                return 0
            lax.fori_loop(0, n, one, 0)

        def wait_rows(n, sem):
            def one(i, _):
                pltpu.make_async_copy(
                    e_ref.at[pl.ds(0, 1), :], gbuf.at[pl.ds(0, 1), :], sem
                ).wait()
                return 0
            lax.fori_loop(0, n, one, 0)

        issue_rows(h, aa * q, 0, gsem.at[0])
        issue_rows(h, aa * q + h, h, gsem.at[1])
        issue_rows(q, bb * q, q, gsem.at[2])

        pl.semaphore_wait(barrier, 3)

        y_sends = []
        for i, (off, ln) in enumerate(((0, h), (h, h), (q, q))):
            wait_rows(ln, gsem.at[i])
            snd = pltpu.make_async_remote_copy(
                src_ref=gbuf.at[pl.ds(off, ln), :],
                dst_ref=fresh_ref.at[pl.ds(off, ln), :],
                send_sem=y_ssem.at[i], recv_sem=y_rsem.at[i],
                device_id=y_nbr, device_id_type=pl.DeviceIdType.MESH,
            )
            snd.start()
            y_sends.append(snd)

        def merge(off, ln, out_off):
            lv = ids_v[pl.ds(out_off, ln), :] - base
            valid = (lv >= 0) & (lv < v_per)
            out_ref[pl.ds(out_off, ln), :] = jnp.where(
                valid,
                gbuf[pl.ds(off, ln), :],
                fresh_ref[pl.ds(off, ln), :],
            )

        fwds = []
        for c in range(2):
            y_sends[c].wait_recv()
            off = c * h
            merge(off, h, aa * q + off)
            for j, nbr in enumerate((x_nbr, z_nbr)):
                k = 2 * c + j
                fwd = pltpu.make_async_remote_copy(
                    src_ref=out_ref.at[pl.ds(aa * q + off, h), :],
                    dst_ref=out_ref.at[pl.ds(aa * q + off, h), :],
                    send_sem=f_ssem.at[k], recv_sem=f_rsem.at[k],
                    device_id=nbr, device_id_type=pl.DeviceIdType.MESH,
                )
                fwd.start()
                fwds.append(fwd)

        y_sends[2].wait_recv()
        merge(q, q, bb * q)

        for fwd in fwds:
            fwd.wait_recv()
        for snd in y_sends:
            snd.wait_send()
        for fwd in fwds:
            fwd.wait_send()

    return pl.pallas_call(
        body,
        out_shape=jax.ShapeDtypeStruct((t, d), jnp.float32),
        in_specs=[
            pl.BlockSpec(memory_space=pltpu.SMEM),
            pl.BlockSpec(memory_space=pltpu.VMEM),
            pl.BlockSpec(memory_space=pl.ANY),
        ],
        out_specs=pl.BlockSpec(memory_space=pltpu.VMEM),
        scratch_shapes=[
            pltpu.VMEM((2 * q, d), jnp.float32),
            pltpu.VMEM((2 * q, d), jnp.float32),
            pltpu.SemaphoreType.DMA((3,)),
            pltpu.SemaphoreType.DMA((3,)),
            pltpu.SemaphoreType.DMA((3,)),
            pltpu.SemaphoreType.DMA((4,)),
            pltpu.SemaphoreType.DMA((4,)),
        ],
        compiler_params=pltpu.CompilerParams(collective_id=0),
    )(ids, ids[:, None], E)


# device time: 15197 ns/iter; 1.3971x vs baseline; 1.3971x over previous
import jax
import jax.numpy as jnp
from jax import lax
from jax.experimental import pallas as pl
from jax.experimental.pallas import tpu as pltpu


def kernel(ids, E):
    v_per, d = E.shape
    t = ids.shape[0]
    q = t // 4
    h = q // 2

    my_y = lax.axis_index("y")
    local = (ids - my_y * v_per).astype(jnp.int32)
    raw = E[jnp.bitwise_and(local, v_per - 1), :]

    def body(raw_ref, lv_ref, out_ref, gbuf, fresh_ref,
             gsem, y_ssem, y_rsem, f_ssem, f_rsem):
        mx = lax.axis_index("x")
        my = lax.axis_index("y")
        mz = lax.axis_index("z")
        y_nbr = (mx, 1 - my, mz)
        x_nbr = (1 - mx, my, mz)
        z_nbr = (mx, my, 1 - mz)
        aa = 2 * mx + mz
        bb = 3 - aa

        barrier = pltpu.get_barrier_semaphore()
        for nbr in (y_nbr, x_nbr, z_nbr):
            pl.semaphore_signal(
                barrier, inc=1, device_id=nbr,
                device_id_type=pl.DeviceIdType.MESH,
            )

        chunks = ((0, h), (h, h), (q, q))
        g_off = (aa * q, aa * q + h, bb * q)
        pulls = []
        for i, (off, ln) in enumerate(chunks):
            cp = pltpu.make_async_copy(
                raw_ref.at[pl.ds(g_off[i], ln), :],
                gbuf.at[pl.ds(off, ln), :],
                gsem.at[i],
            )
            cp.start()
            pulls.append(cp)

        pl.semaphore_wait(barrier, 3)

        y_sends = []
        for i, (off, ln) in enumerate(chunks):
            pulls[i].wait()
            snd = pltpu.make_async_remote_copy(
                src_ref=gbuf.at[pl.ds(off, ln), :],
                dst_ref=fresh_ref.at[pl.ds(off, ln), :],
                send_sem=y_ssem.at[i], recv_sem=y_rsem.at[i],
                device_id=y_nbr, device_id_type=pl.DeviceIdType.MESH,
            )
            snd.start()
            y_sends.append(snd)

        def merge(off, ln, out_off):
            lv = lv_ref[pl.ds(out_off, ln), :]
            valid = (lv >= 0) & (lv < v_per)
            out_ref[pl.ds(out_off, ln), :] = jnp.where(
                valid,
                gbuf[pl.ds(off, ln), :],
                fresh_ref[pl.ds(off, ln), :],
            )

        fwds = []
        for c in range(2):
            y_sends[c].wait_recv()
            off = c * h
            merge(off, h, aa * q + off)
            for j, nbr in enumerate((x_nbr, z_nbr)):
                k = 2 * c + j
                fwd = pltpu.make_async_remote_copy(
                    src_ref=out_ref.at[pl.ds(aa * q + off, h), :],
                    dst_ref=out_ref.at[pl.ds(aa * q + off, h), :],
                    send_sem=f_ssem.at[k], recv_sem=f_rsem.at[k],
                    device_id=nbr, device_id_type=pl.DeviceIdType.MESH,
                )
                fwd.start()
                fwds.append(fwd)

        y_sends[2].wait_recv()
        merge(q, q, bb * q)

        for fwd in fwds:
            fwd.wait_recv()
        for snd in y_sends:
            snd.wait_send()
        for fwd in fwds:
            fwd.wait_send()

    return pl.pallas_call(
        body,
        out_shape=jax.ShapeDtypeStruct((t, d), jnp.float32),
        in_specs=[
            pl.BlockSpec(memory_space=pl.ANY),
            pl.BlockSpec(memory_space=pltpu.VMEM),
        ],
        out_specs=pl.BlockSpec(memory_space=pltpu.VMEM),
        scratch_shapes=[
            pltpu.VMEM((2 * q, d), jnp.float32),
            pltpu.VMEM((2 * q, d), jnp.float32),
            pltpu.SemaphoreType.DMA((3,)),
            pltpu.SemaphoreType.DMA((3,)),
            pltpu.SemaphoreType.DMA((3,)),
            pltpu.SemaphoreType.DMA((4,)),
            pltpu.SemaphoreType.DMA((4,)),
        ],
        compiler_params=pltpu.CompilerParams(collective_id=0),
    )(raw, local[:, None])
